# baseline (speedup 1.0000x reference)
import jax
import jax.numpy as jnp
from jax import lax
from jax.experimental import pallas as pl
from jax.experimental.pallas import tpu as pltpu

BATCH = 16384
MAX_LEN = 150
PAD_LEN = 152
FLAT_ROWS = BATCH * PAD_LEN // 128
GRID = 8
BLOCK_ROWS = FLAT_ROWS // GRID


def _body(x_ref, ids_ref, mask_ref, type_ref):
    x = x_ref[...]
    ids_ref[...] = x
    mask_ref[...] = jnp.where(x == 0, 0, 1).astype(jnp.int32)
    type_ref[...] = jnp.zeros_like(x)


def kernel(inputs):
    xp = lax.pad(inputs, jnp.int32(0), ((0, 0, 0), (0, PAD_LEN - MAX_LEN, 0)))
    flat = xp.reshape(FLAT_ROWS, 128)
    spec = pl.BlockSpec((BLOCK_ROWS, 128), lambda i: (i, 0))
    out_shape = jax.ShapeDtypeStruct((FLAT_ROWS, 128), jnp.int32)
    outs = pl.pallas_call(
        _body,
        grid=(GRID,),
        in_specs=[spec],
        out_specs=[spec, spec, spec],
        out_shape=[out_shape, out_shape, out_shape],
        compiler_params=pltpu.CompilerParams(
            dimension_semantics=("arbitrary",),
        ),
    )(flat)
    return tuple(
        o.reshape(BATCH, PAD_LEN)[:, :MAX_LEN] for o in outs)


# speedup vs baseline: 10.9578x; 10.9578x over previous
import jax
import jax.numpy as jnp
from jax.experimental import pallas as pl
from jax.experimental.pallas import tpu as pltpu

BATCH = 16384
MAX_LEN = 150
GRID = 8
BLOCK_COLS = BATCH // GRID


def _body(x_ref, ids_ref, mask_ref, type_ref):
    x = x_ref[...]
    ids_ref[...] = x
    mask_ref[...] = jnp.where(x == 0, 0, 1).astype(jnp.int32)
    type_ref[...] = jnp.zeros_like(x)


def kernel(inputs):
    xt = inputs.T
    spec = pl.BlockSpec((MAX_LEN, BLOCK_COLS), lambda i: (0, i))
    out_shape = jax.ShapeDtypeStruct((MAX_LEN, BATCH), jnp.int32)
    ids, mask, type_ids = pl.pallas_call(
        _body,
        grid=(GRID,),
        in_specs=[spec],
        out_specs=[spec, spec, spec],
        out_shape=[out_shape, out_shape, out_shape],
        compiler_params=pltpu.CompilerParams(
            dimension_semantics=("arbitrary",),
        ),
    )(xt)
    return (ids.T, mask.T, type_ids.T)


# speedup vs baseline: 12.0197x; 1.0969x over previous
import jax
import jax.numpy as jnp
from jax.experimental import pallas as pl
from jax.experimental.pallas import tpu as pltpu

BATCH = 16384
MAX_LEN = 150
GRID = 4
BLOCK_COLS = BATCH // GRID


def _body(x_ref, ids_ref, mask_ref, type_ref):
    x = x_ref[...]
    ids_ref[...] = x
    mask_ref[...] = jnp.where(x == 0, 0, 1).astype(jnp.int32)
    type_ref[...] = jnp.zeros_like(x)


def kernel(inputs):
    xt = inputs.T
    spec = pl.BlockSpec((MAX_LEN, BLOCK_COLS), lambda i: (0, i))
    out_shape = jax.ShapeDtypeStruct((MAX_LEN, BATCH), jnp.int32)
    ids, mask, type_ids = pl.pallas_call(
        _body,
        grid=(GRID,),
        in_specs=[spec],
        out_specs=[spec, spec, spec],
        out_shape=[out_shape, out_shape, out_shape],
        compiler_params=pltpu.CompilerParams(
            dimension_semantics=("arbitrary",),
        ),
    )(xt)
    return (ids.T, mask.T, type_ids.T)
